# baseline (speedup 1.0000x reference)
import functools

import jax
import jax.numpy as jnp
from jax import lax
from jax.experimental import pallas as pl
from jax.experimental.pallas import tpu as pltpu
from jax.experimental.pallas import tpu_sc as plsc

B = 4096
CTX = 20
EMB = 64
EMBP = 128
LANES = 16

_info = plsc.get_sparse_core_info()
_NC, _NS = _info.num_cores, _info.num_subcores
_NW = _NC * _NS
_BPW = B // _NW


def _sc_gather_sum(xT, table_pad):
    mesh = plsc.VectorSubcoreMesh(core_axis_name="c", subcore_axis_name="s")

    @functools.partial(
        pl.kernel,
        out_type=jax.ShapeDtypeStruct((B, EMBP), jnp.float32),
        mesh=mesh,
        compiler_params=pltpu.CompilerParams(use_tc_tiling_on_sc=True),
        scratch_types=[
            pltpu.VMEM((CTX, _BPW), jnp.int32),
            pltpu.VMEM((_BPW, EMBP), jnp.float32),
            pltpu.SemaphoreType.DMA,
            pltpu.SemaphoreType.DMA,
        ],
    )
    def sc_kernel(xT_hbm, table_hbm, out_hbm, idx_v, acc_v, sem_acc, sem_add):
        wid = lax.axis_index("s") * _NC + lax.axis_index("c")
        base = wid * _BPW

        pltpu.sync_copy(xT_hbm.at[:, pl.ds(base, _BPW)], idx_v)

        pltpu.async_copy(table_hbm.at[idx_v.at[0]], acc_v, sem_acc).wait()
        sems = (sem_acc, sem_add)
        adds = [pltpu.async_copy(table_hbm.at[idx_v.at[j]], acc_v,
                                 sems[j % 2], add=True)
                for j in range(1, CTX)]
        for d in adds:
            d.wait()
        pltpu.sync_copy(acc_v, out_hbm.at[pl.ds(base, _BPW)])

    return sc_kernel(xT, table_pad)


_VB = 1024


def _tc_matmul_bias(WT, mean, brow):
    V = WT.shape[1]
    nv = pl.cdiv(V, _VB)

    def mm_body(wT_ref, mean_ref, b_ref, out_ref):
        acc = lax.dot_general(wT_ref[...],
                              mean_ref[...] * jnp.float32(1.0 / CTX),
                              (((0,), (1,)), ((), ())),
                              preferred_element_type=jnp.float32)
        out_ref[...] = acc + jnp.transpose(b_ref[...])

    return pl.pallas_call(
        mm_body,
        grid=(nv,),
        in_specs=[
            pl.BlockSpec((EMB, _VB), lambda j: (0, j)),
            pl.BlockSpec((B, EMB), lambda j: (0, 0)),
            pl.BlockSpec((1, _VB), lambda j: (0, j)),
        ],
        out_specs=pl.BlockSpec((_VB, B), lambda j: (j, 0)),
        out_shape=jax.ShapeDtypeStruct((V, B), jnp.float32),
        compiler_params=pltpu.CompilerParams(
            dimension_semantics=("arbitrary",),
        ),
    )(WT, mean, brow)


def kernel(x, emb_table, W, b):
    xT = jnp.transpose(x.astype(jnp.int32))
    table_pad = jnp.pad(emb_table, ((0, 0), (0, EMBP - EMB)))
    sums = _sc_gather_sum(xT, table_pad)
    scoresT = _tc_matmul_bias(W.T, sums[:, :EMB], b.reshape(1, -1))
    return scoresT.T


# speedup vs baseline: 1.0010x; 1.0010x over previous
import functools

import jax
import jax.numpy as jnp
from jax import lax
from jax.experimental import pallas as pl
from jax.experimental.pallas import tpu as pltpu
from jax.experimental.pallas import tpu_sc as plsc

B = 4096
CTX = 20
EMB = 64
EMBP = 128
LANES = 16

_info = plsc.get_sparse_core_info()
_NC, _NS = _info.num_cores, _info.num_subcores
_NW = _NC * _NS
_BPW = B // _NW


def _sc_gather_sum(xT, table_pad):
    mesh = plsc.VectorSubcoreMesh(core_axis_name="c", subcore_axis_name="s")

    @functools.partial(
        pl.kernel,
        out_type=jax.ShapeDtypeStruct((B, EMBP), jnp.float32),
        mesh=mesh,
        compiler_params=pltpu.CompilerParams(use_tc_tiling_on_sc=True),
        scratch_types=[
            pltpu.VMEM((CTX, _BPW), jnp.int32),
            pltpu.VMEM((_BPW, EMBP), jnp.float32),
            pltpu.SemaphoreType.DMA,
            pltpu.SemaphoreType.DMA,
        ],
    )
    def sc_kernel(xT_hbm, table_hbm, out_hbm, idx_v, acc_v, sem_acc, sem_add):
        wid = lax.axis_index("s") * _NC + lax.axis_index("c")
        base = wid * _BPW

        pltpu.sync_copy(xT_hbm.at[:, pl.ds(base, _BPW)], idx_v)

        pltpu.async_copy(table_hbm.at[idx_v.at[0]], acc_v, sem_acc).wait()
        adds = [pltpu.async_copy(table_hbm.at[idx_v.at[j]], acc_v, sem_add,
                                 add=True)
                for j in range(1, CTX)]
        for d in adds:
            d.wait()
        pltpu.sync_copy(acc_v, out_hbm.at[pl.ds(base, _BPW)])

    return sc_kernel(xT, table_pad)


_VB = 1024


def _tc_matmul_bias(WT, mean, brow):
    V = WT.shape[1]
    nv = pl.cdiv(V, _VB)

    def mm_body(wT_ref, mean_ref, b_ref, out_ref):
        acc = lax.dot_general(wT_ref[...],
                              mean_ref[...] * jnp.float32(1.0 / CTX),
                              (((0,), (1,)), ((), ())),
                              preferred_element_type=jnp.float32)
        out_ref[...] = acc + jnp.transpose(b_ref[...])

    return pl.pallas_call(
        mm_body,
        grid=(nv,),
        in_specs=[
            pl.BlockSpec((EMB, _VB), lambda j: (0, j)),
            pl.BlockSpec((B, EMB), lambda j: (0, 0)),
            pl.BlockSpec((1, _VB), lambda j: (0, j)),
        ],
        out_specs=pl.BlockSpec((_VB, B), lambda j: (j, 0)),
        out_shape=jax.ShapeDtypeStruct((V, B), jnp.float32),
        compiler_params=pltpu.CompilerParams(
            dimension_semantics=("arbitrary",),
        ),
    )(WT, mean, brow)


def kernel(x, emb_table, W, b):
    xT = jnp.transpose(x.astype(jnp.int32))
    table_pad = jnp.pad(emb_table, ((0, 0), (0, EMBP - EMB)))
    sums = _sc_gather_sum(xT, table_pad)
    scoresT = _tc_matmul_bias(W.T, sums[:, :EMB], b.reshape(1, -1))
    return scoresT.T
